# Initial kernel scaffold; baseline (speedup 1.0000x reference)
import jax
import jax.numpy as jnp
from jax.experimental import pallas as pl


def kernel(preds, low_feats, high_feats_1, high_feats_2, high_feats_3, unlabeled_ROIs, weight):
    raise NotImplementedError("write your pallas kernel here")


# speedup vs baseline: 434.9150x; 434.9150x over previous
import functools

import jax
import jax.numpy as jnp
from jax import lax
from jax.experimental import pallas as pl
from jax.experimental.pallas import tpu as pltpu
from jax.experimental.pallas import tpu_sc as plsc

_SIGMA = 0.02
_ITERS = 4
_H = 128
_W = 128
_N = _H * _W
_M = 2 * _N
_SEN = _W - 1
_NTREE = 16


def _dist_body(nc, f_ref, out_ref):
    c = pl.program_id(1)

    @pl.when(c == 0)
    def _():
        out_ref[...] = jnp.zeros_like(out_ref)

    x = f_ref[0]
    dh = x - jnp.roll(x, -1, axis=2)
    dv = x - jnp.roll(x, -1, axis=1)
    acc = out_ref[0]
    out_ref[0] = jnp.stack([acc[0] + jnp.sum(dh * dh, axis=0),
                            acc[1] + jnp.sum(dv * dv, axis=0)])

    @pl.when(c == nc - 1)
    def _():
        d = out_ref[0]
        col = lax.broadcasted_iota(jnp.int32, (_H, _W), 1)
        row = lax.broadcasted_iota(jnp.int32, (_H, _W), 0)
        inf = jnp.float32(jnp.inf)
        d0 = jnp.where(col == _W - 1, inf, d[0])
        d1 = jnp.where(row == _H - 1, inf, d[1])
        out_ref[0] = jnp.stack([d0, d1])


def _edge_dists(f):
    b, ch, _, _ = f.shape
    chunk = ch if ch <= 64 else 64
    nc = ch // chunk
    return pl.pallas_call(
        functools.partial(_dist_body, nc),
        grid=(b, nc),
        in_specs=[pl.BlockSpec((1, chunk, _H, _W), lambda i, c: (i, c, 0, 0))],
        out_specs=pl.BlockSpec((1, 2, _H, _W), lambda i, c: (i, 0, 0, 0)),
        out_shape=jax.ShapeDtypeStruct((b, 2, _H, _W), jnp.float32),
    )(f)


_NBLK_N = _N // 16
_NBLK_M = _M // 16


def _lane_iota():
    return lax.iota(jnp.int32, 16)


def _scatter_min(minidx_ref, w_ref, cidx, e, we, active):
    def cond(m):
        return jnp.sum(m) > 0

    def body(m):
        mask = m != 0
        cur = plsc.load_gather(minidx_ref, [cidx])
        wcur = plsc.load_gather(w_ref, [cur])
        better = mask & ((we < wcur) | ((we == wcur) & (e < cur)))
        plsc.store_scatter(minidx_ref, [cidx], e, mask=better)
        return better.astype(jnp.int32)

    lax.while_loop(cond, body, active.astype(jnp.int32))


def _mst_body(w_hbm, out_hbm, w_v, comp_v, minidx_v, ptr_v, mark_v, sem):
    wid = lax.axis_index("s") * 2 + lax.axis_index("c")

    @pl.when(wid < _NTREE)
    def _run():
        pltpu.sync_copy(w_hbm.at[wid], w_v)
        iota = _lane_iota()
        sen_vec = jnp.full((16,), _SEN, jnp.int32)

        def init_blk(i, _):
            ds = pl.ds(i * 16, 16)
            comp_v[ds] = i * 16 + iota
            minidx_v[ds] = sen_vec
            mark_v[ds] = jnp.zeros((16,), jnp.int32)
            mark_v[pl.ds(_N + i * 16, 16)] = jnp.zeros((16,), jnp.int32)
            return 0

        lax.fori_loop(0, _NBLK_N, init_blk, 0)

        def edge_blk(i, _):
            e = i * 16 + iota
            is_h = e < _N
            u = jnp.where(is_h, e, e - _N)
            v = jnp.where(is_h, e + 1, u + _W)
            v = jnp.minimum(v, _N - 1)
            cu = plsc.load_gather(comp_v, [u])
            cv = plsc.load_gather(comp_v, [v])
            we = w_v[pl.ds(i * 16, 16)]
            active = cu != cv
            _scatter_min(minidx_v, w_v, cu, e, we, active)
            _scatter_min(minidx_v, w_v, cv, e, we, active)
            return 0

        def link_blk(i, cnt):
            ds = pl.ds(i * 16, 16)
            c = i * 16 + iota
            e = minidx_v[ds]
            has = e != _SEN
            plsc.store_scatter(mark_v, [e], jnp.ones((16,), jnp.int32),
                               mask=has)
            is_h = e < _N
            u = jnp.where(is_h, e, e - _N)
            v = jnp.where(is_h, e + 1, u + _W)
            v = jnp.minimum(v, _N - 1)
            cu = plsc.load_gather(comp_v, [u])
            cv = plsc.load_gather(comp_v, [v])
            other = jnp.where(cu == c, cv, cu)
            ptr_v[ds] = jnp.where(has, other, c)
            minidx_v[ds] = sen_vec
            return cnt + jnp.sum(has.astype(jnp.int32))

        def cycle_blk(i, _):
            ds = pl.ds(i * 16, 16)
            c = i * 16 + iota
            p = ptr_v[ds]
            pp = plsc.load_gather(ptr_v, [p])
            ptr_v[ds] = jnp.where((pp == c) & (c < p), c, p)
            return 0

        def jump_blk(i, _):
            ds = pl.ds(i * 16, 16)
            p = ptr_v[ds]
            ptr_v[ds] = plsc.load_gather(ptr_v, [p])
            return 0

        def jump_pass(_, x):
            return lax.fori_loop(0, _NBLK_N, jump_blk, x)

        def comp_blk(i, _):
            ds = pl.ds(i * 16, 16)
            comp_v[ds] = plsc.load_gather(ptr_v, [comp_v[ds]])
            return 0

        def round_body(r, _):
            lax.fori_loop(0, _NBLK_M, edge_blk, 0)
            lax.fori_loop(0, _NBLK_N, link_blk, 0)
            lax.fori_loop(0, _NBLK_N, cycle_blk, 0)
            lax.fori_loop(0, 14 - r, jump_pass, 0)
            lax.fori_loop(0, _NBLK_N, comp_blk, 0)
            return 0

        lax.fori_loop(0, 14, round_body, 0)
        pltpu.sync_copy(mark_v, out_hbm.at[wid])


def _mst_masks(w_all):
    mesh = plsc.VectorSubcoreMesh(core_axis_name="c", subcore_axis_name="s",
                                  num_cores=2, num_subcores=16)
    k = functools.partial(
        pl.kernel,
        out_type=jax.ShapeDtypeStruct((_NTREE, _M), jnp.int32),
        mesh=mesh,
        scratch_types=[
            pltpu.VMEM((_M,), jnp.float32),
            pltpu.VMEM((_N,), jnp.int32),
            pltpu.VMEM((_N,), jnp.int32),
            pltpu.VMEM((_N,), jnp.int32),
            pltpu.VMEM((_M,), jnp.int32),
            pltpu.SemaphoreType.DMA,
        ],
        compiler_params=pltpu.CompilerParams(needs_layout_passes=False),
    )(_mst_body)
    return k(w_all)


def _agg3(x, ewh, ewv):
    t = ewh * x
    s = ewv * x
    return (ewh * jnp.roll(x, -1, axis=2) + jnp.roll(t, 1, axis=2)
            + ewv * jnp.roll(x, -1, axis=1) + jnp.roll(s, 1, axis=1))


def _agg2(x, ewh, ewv):
    t = ewh * x
    s = ewv * x
    return (ewh * jnp.roll(x, -1, axis=1) + jnp.roll(t, 1, axis=1)
            + ewv * jnp.roll(x, -1, axis=0) + jnp.roll(s, 1, axis=0))


def _tree_filter(feat, ewh, ewv):
    num = feat
    den = jnp.ones((_H, _W), jnp.float32)
    for _ in range(_ITERS):
        num = feat + _agg3(num, ewh, ewv)
        den = 1.0 + _agg2(den, ewh, ewv)
    return num / den[None]


def _loss_body(preds_ref, dl_ref, d1_ref, d2_ref, d3_ref,
               ml_ref, m1_ref, m2_ref, m3_ref, roi_ref,
               loss_ref, rsum_ref):
    logits = preds_ref[0]
    m = jnp.max(logits, axis=0, keepdims=True)
    ex = jnp.exp(logits - m)
    p = ex / jnp.sum(ex, axis=0, keepdims=True)

    def ew(d_ref, m_ref):
        d = d_ref[0]
        mk = m_ref[0]
        wh = jnp.exp(-d[0] / _SIGMA) * mk[0]
        wv = jnp.exp(-d[1] / _SIGMA) * mk[1]
        return wh, wv

    lh, lv = ew(dl_ref, ml_ref)
    a_s = _tree_filter(p, lh, lv)
    roi = roi_ref[0, 0]
    loss = jnp.float32(0.0)
    for dr, mr in ((d1_ref, m1_ref), (d2_ref, m2_ref), (d3_ref, m3_ref)):
        wh, wv = ew(dr, mr)
        ask = _tree_filter(a_s, wh, wv)
        loss = loss + jnp.sum(roi[None] * jnp.abs(p - ask))
    loss_ref[...] = jnp.full((1, 8, 128), loss)
    rsum_ref[...] = jnp.full((1, 8, 128), jnp.sum(roi))


def _loss_parts(preds, dists, masks, roi):
    b = preds.shape[0]
    spec4 = pl.BlockSpec((1, 2, _H, _W), lambda i: (i, 0, 0, 0))
    outs = pl.pallas_call(
        _loss_body,
        grid=(b,),
        in_specs=[pl.BlockSpec((1, preds.shape[1], _H, _W),
                               lambda i: (i, 0, 0, 0))]
        + [spec4] * 8
        + [pl.BlockSpec((1, 1, _H, _W), lambda i: (i, 0, 0, 0))],
        out_specs=[pl.BlockSpec((1, 8, 128), lambda i: (i, 0, 0))] * 2,
        out_shape=[jax.ShapeDtypeStruct((b, 8, 128), jnp.float32)] * 2,
    )(preds, *dists, *masks, roi)
    return outs


def _resize(x, h, w, method):
    return jax.image.resize(x, (x.shape[0], x.shape[1], h, w), method=method)


@jax.jit
def kernel(preds, low_feats, high_feats_1, high_feats_2, high_feats_3,
           unlabeled_ROIs, weight):
    b, c, h, w = preds.shape
    low_r = _resize(low_feats, h, w, 'bilinear')
    h1 = _resize(high_feats_1, h, w, 'bilinear')
    h2 = _resize(high_feats_2, h, w, 'bilinear')
    h3 = _resize(high_feats_3, h, w, 'bilinear')
    roi = jax.image.resize(unlabeled_ROIs[:, None].astype(jnp.float32),
                           (b, 1, h, w), method='nearest')

    d_low = _edge_dists(low_r)
    d_1 = _edge_dists(h1)
    d_2 = _edge_dists(h2)
    d_3 = _edge_dists(h3)

    w_all = jnp.concatenate([d_low, d_1, d_2, d_3], axis=0).reshape(_NTREE, _M)
    marks = _mst_masks(w_all).astype(jnp.float32).reshape(_NTREE, 2, _H, _W)
    m_low, m_1, m_2, m_3 = (marks[0:4], marks[4:8], marks[8:12], marks[12:16])

    loss_b, rsum_b = _loss_parts(
        preds, (d_low, d_1, d_2, d_3), (m_low, m_1, m_2, m_3), roi)
    loss = jnp.sum(loss_b[:, 0, 0])
    n = jnp.sum(rsum_b[:, 0, 0])
    loss = jnp.where(n > 0, loss / n, loss)
    return jnp.float32(weight) * loss


# speedup vs baseline: 656.3077x; 1.5090x over previous
import functools

import jax
import jax.numpy as jnp
from jax import lax
from jax.experimental import pallas as pl
from jax.experimental.pallas import tpu as pltpu
from jax.experimental.pallas import tpu_sc as plsc

_SIGMA = 0.02
_ITERS = 4
_H = 128
_W = 128
_N = _H * _W
_M = 2 * _N
_SEN = _W - 1
_NTREE = 16


def _dist_body(nc, f_ref, out_ref):
    c = pl.program_id(1)

    @pl.when(c == 0)
    def _():
        out_ref[...] = jnp.zeros_like(out_ref)

    x = f_ref[0]
    dh = x - jnp.roll(x, -1, axis=2)
    dv = x - jnp.roll(x, -1, axis=1)
    acc = out_ref[0]
    out_ref[0] = jnp.stack([acc[0] + jnp.sum(dh * dh, axis=0),
                            acc[1] + jnp.sum(dv * dv, axis=0)])

    @pl.when(c == nc - 1)
    def _():
        d = out_ref[0]
        col = lax.broadcasted_iota(jnp.int32, (_H, _W), 1)
        row = lax.broadcasted_iota(jnp.int32, (_H, _W), 0)
        inf = jnp.float32(jnp.inf)
        d0 = jnp.where(col == _W - 1, inf, d[0])
        d1 = jnp.where(row == _H - 1, inf, d[1])
        out_ref[0] = jnp.stack([d0, d1])


def _edge_dists(f):
    b, ch, _, _ = f.shape
    chunk = ch if ch <= 64 else 64
    nc = ch // chunk
    return pl.pallas_call(
        functools.partial(_dist_body, nc),
        grid=(b, nc),
        in_specs=[pl.BlockSpec((1, chunk, _H, _W), lambda i, c: (i, c, 0, 0))],
        out_specs=pl.BlockSpec((1, 2, _H, _W), lambda i, c: (i, 0, 0, 0)),
        out_shape=jax.ShapeDtypeStruct((b, 2, _H, _W), jnp.float32),
    )(f)


_NBLK_N = _N // 16
_NBLK_M = _M // 16


def _lane_iota():
    return lax.iota(jnp.int32, 16)


def _scatter_min(minidx_ref, w_ref, cidx, e, we, active):
    def cond(m):
        return jnp.sum(m) > 0

    def body(m):
        mask = m != 0
        cur = plsc.load_gather(minidx_ref, [cidx])
        wcur = plsc.load_gather(w_ref, [cur])
        better = mask & ((we < wcur) | ((we == wcur) & (e < cur)))
        plsc.store_scatter(minidx_ref, [cidx], e, mask=better)
        return better.astype(jnp.int32)

    lax.while_loop(cond, body, active.astype(jnp.int32))


def _mst_body(w_hbm, out_hbm, w_v, comp_v, minidx_v, ptr_v, mark_v, sem):
    wid = lax.axis_index("s") * 2 + lax.axis_index("c")

    @pl.when(wid < _NTREE)
    def _run():
        pltpu.sync_copy(w_hbm.at[wid], w_v)
        iota = _lane_iota()
        sen_vec = jnp.full((16,), _SEN, jnp.int32)
        zero_vec = jnp.zeros((16,), jnp.int32)

        def init_blk(i, _):
            ds = pl.ds(i * 16, 16)
            comp_v[ds] = i * 16 + iota
            minidx_v[ds] = sen_vec
            mark_v[ds] = zero_vec
            mark_v[pl.ds(_N + i * 16, 16)] = zero_vec
            return 0

        lax.fori_loop(0, _NBLK_N, init_blk, 0)

        def slack_blk(i, _):
            comp_v[pl.ds(_N + i * 16, 16)] = jnp.full((16,), _N - 1, jnp.int32)
            return 0

        lax.fori_loop(0, _W // 16, slack_blk, 0)

        def edge_h_blk(i, _):
            base = i * 16
            e = base + iota
            cu = comp_v[pl.ds(base, 16)]
            cv = comp_v[pl.ds(base + 1, 16)]
            we = w_v[pl.ds(base, 16)]
            active = cu != cv
            _scatter_min(minidx_v, w_v, cu, e, we, active)
            _scatter_min(minidx_v, w_v, cv, e, we, active)
            return 0

        def edge_v_blk(i, _):
            base = i * 16
            e = _N + base + iota
            cu = comp_v[pl.ds(base, 16)]
            cv = comp_v[pl.ds(base + _W, 16)]
            we = w_v[pl.ds(_N + base, 16)]
            active = cu != cv
            _scatter_min(minidx_v, w_v, cu, e, we, active)
            _scatter_min(minidx_v, w_v, cv, e, we, active)
            return 0

        def link_blk(i, acc):
            ds = pl.ds(i * 16, 16)
            c = i * 16 + iota
            e = minidx_v[ds]
            has = e != _SEN
            plsc.store_scatter(mark_v, [e], jnp.ones((16,), jnp.int32),
                               mask=has)
            is_h = e < _N
            u = jnp.where(is_h, e, e - _N)
            v = jnp.where(is_h, e + 1, u + _W)
            v = jnp.minimum(v, _N - 1)
            cu = plsc.load_gather(comp_v, [u])
            cv = plsc.load_gather(comp_v, [v])
            other = jnp.where(cu == c, cv, cu)
            ptr_v[ds] = jnp.where(has, other, c)
            minidx_v[ds] = sen_vec
            return acc | has.astype(jnp.int32)

        def cycle_blk(i, _):
            ds = pl.ds(i * 16, 16)
            c = i * 16 + iota
            p = ptr_v[ds]
            pp = plsc.load_gather(ptr_v, [p])
            ptr_v[ds] = jnp.where((pp == c) & (c < p), c, p)
            return 0

        def jump_blk(i, acc):
            ds = pl.ds(i * 16, 16)
            p = ptr_v[ds]
            pp = plsc.load_gather(ptr_v, [p])
            ptr_v[ds] = pp
            return acc | (pp != p).astype(jnp.int32)

        def jump_pass(_, changed):
            nj = jnp.where(changed > 0, _NBLK_N, 0)
            acc = lax.fori_loop(0, nj, jump_blk, zero_vec)
            return jnp.sum(acc)

        def comp_blk(i, _):
            ds = pl.ds(i * 16, 16)
            comp_v[ds] = plsc.load_gather(ptr_v, [comp_v[ds]])
            return 0

        def round_body(r, go):
            ne = jnp.where(go > 0, _NBLK_N, 0)
            lax.fori_loop(0, ne, edge_h_blk, 0)
            lax.fori_loop(0, ne, edge_v_blk, 0)
            merged = jnp.sum(lax.fori_loop(0, ne, link_blk, zero_vec))
            lax.fori_loop(0, ne, cycle_blk, 0)
            lax.fori_loop(0, jnp.where(merged > 0, 14 - r, 0), jump_pass, 1)
            lax.fori_loop(0, ne, comp_blk, 0)
            return jnp.where(merged > 0, jnp.int32(1), jnp.int32(0))

        lax.fori_loop(0, 14, round_body, jnp.int32(1))
        pltpu.sync_copy(mark_v, out_hbm.at[wid])


def _mst_masks(w_all):
    mesh = plsc.VectorSubcoreMesh(core_axis_name="c", subcore_axis_name="s",
                                  num_cores=2, num_subcores=16)
    k = functools.partial(
        pl.kernel,
        out_type=jax.ShapeDtypeStruct((_NTREE, _M), jnp.int32),
        mesh=mesh,
        scratch_types=[
            pltpu.VMEM((_M,), jnp.float32),
            pltpu.VMEM((_N + _W,), jnp.int32),
            pltpu.VMEM((_N,), jnp.int32),
            pltpu.VMEM((_N,), jnp.int32),
            pltpu.VMEM((_M,), jnp.int32),
            pltpu.SemaphoreType.DMA,
        ],
        compiler_params=pltpu.CompilerParams(needs_layout_passes=False),
    )(_mst_body)
    return k(w_all)


def _agg3(x, ewh, ewv):
    t = ewh * x
    s = ewv * x
    return (ewh * jnp.roll(x, -1, axis=2) + jnp.roll(t, 1, axis=2)
            + ewv * jnp.roll(x, -1, axis=1) + jnp.roll(s, 1, axis=1))


def _agg2(x, ewh, ewv):
    t = ewh * x
    s = ewv * x
    return (ewh * jnp.roll(x, -1, axis=1) + jnp.roll(t, 1, axis=1)
            + ewv * jnp.roll(x, -1, axis=0) + jnp.roll(s, 1, axis=0))


def _tree_filter(feat, ewh, ewv):
    num = feat
    den = jnp.ones((_H, _W), jnp.float32)
    for _ in range(_ITERS):
        num = feat + _agg3(num, ewh, ewv)
        den = 1.0 + _agg2(den, ewh, ewv)
    return num / den[None]


def _loss_body(preds_ref, dl_ref, d1_ref, d2_ref, d3_ref,
               ml_ref, m1_ref, m2_ref, m3_ref, roi_ref,
               loss_ref, rsum_ref):
    logits = preds_ref[0]
    m = jnp.max(logits, axis=0, keepdims=True)
    ex = jnp.exp(logits - m)
    p = ex / jnp.sum(ex, axis=0, keepdims=True)

    def ew(d_ref, m_ref):
        d = d_ref[0]
        mk = m_ref[0]
        wh = jnp.exp(-d[0] / _SIGMA) * mk[0]
        wv = jnp.exp(-d[1] / _SIGMA) * mk[1]
        return wh, wv

    lh, lv = ew(dl_ref, ml_ref)
    a_s = _tree_filter(p, lh, lv)
    roi = roi_ref[0, 0]
    loss = jnp.float32(0.0)
    for dr, mr in ((d1_ref, m1_ref), (d2_ref, m2_ref), (d3_ref, m3_ref)):
        wh, wv = ew(dr, mr)
        ask = _tree_filter(a_s, wh, wv)
        loss = loss + jnp.sum(roi[None] * jnp.abs(p - ask))
    loss_ref[...] = jnp.full((1, 8, 128), loss)
    rsum_ref[...] = jnp.full((1, 8, 128), jnp.sum(roi))


def _loss_parts(preds, dists, masks, roi):
    b = preds.shape[0]
    spec4 = pl.BlockSpec((1, 2, _H, _W), lambda i: (i, 0, 0, 0))
    outs = pl.pallas_call(
        _loss_body,
        grid=(b,),
        in_specs=[pl.BlockSpec((1, preds.shape[1], _H, _W),
                               lambda i: (i, 0, 0, 0))]
        + [spec4] * 8
        + [pl.BlockSpec((1, 1, _H, _W), lambda i: (i, 0, 0, 0))],
        out_specs=[pl.BlockSpec((1, 8, 128), lambda i: (i, 0, 0))] * 2,
        out_shape=[jax.ShapeDtypeStruct((b, 8, 128), jnp.float32)] * 2,
    )(preds, *dists, *masks, roi)
    return outs


def _resize(x, h, w, method):
    return jax.image.resize(x, (x.shape[0], x.shape[1], h, w), method=method)


@jax.jit
def kernel(preds, low_feats, high_feats_1, high_feats_2, high_feats_3,
           unlabeled_ROIs, weight):
    b, c, h, w = preds.shape
    low_r = _resize(low_feats, h, w, 'bilinear')
    h1 = _resize(high_feats_1, h, w, 'bilinear')
    h2 = _resize(high_feats_2, h, w, 'bilinear')
    h3 = _resize(high_feats_3, h, w, 'bilinear')
    roi = jax.image.resize(unlabeled_ROIs[:, None].astype(jnp.float32),
                           (b, 1, h, w), method='nearest')

    d_low = _edge_dists(low_r)
    d_1 = _edge_dists(h1)
    d_2 = _edge_dists(h2)
    d_3 = _edge_dists(h3)

    w_all = jnp.concatenate([d_low, d_1, d_2, d_3], axis=0).reshape(_NTREE, _M)
    marks = _mst_masks(w_all).astype(jnp.float32).reshape(_NTREE, 2, _H, _W)
    m_low, m_1, m_2, m_3 = (marks[0:4], marks[4:8], marks[8:12], marks[12:16])

    loss_b, rsum_b = _loss_parts(
        preds, (d_low, d_1, d_2, d_3), (m_low, m_1, m_2, m_3), roi)
    loss = jnp.sum(loss_b[:, 0, 0])
    n = jnp.sum(rsum_b[:, 0, 0])
    loss = jnp.where(n > 0, loss / n, loss)
    return jnp.float32(weight) * loss


# speedup vs baseline: 750.7668x; 1.1439x over previous
import functools

import jax
import jax.numpy as jnp
from jax import lax
from jax.experimental import pallas as pl
from jax.experimental.pallas import tpu as pltpu
from jax.experimental.pallas import tpu_sc as plsc

_SIGMA = 0.02
_ITERS = 4
_H = 128
_W = 128
_N = _H * _W
_M = 2 * _N
_SEN = _W - 1
_NTREE = 16


def _dist_body(nc, f_ref, out_ref):
    c = pl.program_id(1)

    @pl.when(c == 0)
    def _():
        out_ref[...] = jnp.zeros_like(out_ref)

    x = f_ref[0]
    dh = x - jnp.roll(x, -1, axis=2)
    dv = x - jnp.roll(x, -1, axis=1)
    acc = out_ref[0]
    out_ref[0] = jnp.stack([acc[0] + jnp.sum(dh * dh, axis=0),
                            acc[1] + jnp.sum(dv * dv, axis=0)])

    @pl.when(c == nc - 1)
    def _():
        d = out_ref[0]
        col = lax.broadcasted_iota(jnp.int32, (_H, _W), 1)
        row = lax.broadcasted_iota(jnp.int32, (_H, _W), 0)
        inf = jnp.float32(jnp.inf)
        d0 = jnp.where(col == _W - 1, inf, d[0])
        d1 = jnp.where(row == _H - 1, inf, d[1])
        out_ref[0] = jnp.stack([d0, d1])


def _edge_dists(f):
    b, ch, _, _ = f.shape
    chunk = ch if ch <= 64 else 64
    nc = ch // chunk
    return pl.pallas_call(
        functools.partial(_dist_body, nc),
        grid=(b, nc),
        in_specs=[pl.BlockSpec((1, chunk, _H, _W), lambda i, c: (i, c, 0, 0))],
        out_specs=pl.BlockSpec((1, 2, _H, _W), lambda i, c: (i, 0, 0, 0)),
        out_shape=jax.ShapeDtypeStruct((b, 2, _H, _W), jnp.float32),
    )(f)


_NBLK_N = _N // 16
_NBLK_M = _M // 16


def _lane_iota():
    return lax.iota(jnp.int32, 16)


def _scatter_min(minidx_ref, w_ref, cidx, e, we, active):
    def step(m):
        cur = plsc.load_gather(minidx_ref, [cidx])
        wcur = plsc.load_gather(w_ref, [cur])
        better = m & ((we < wcur) | ((we == wcur) & (e < cur)))
        plsc.store_scatter(minidx_ref, [cidx], e, mask=better)
        return better

    def cond(m):
        return jnp.sum(m) > 0

    def body(m):
        return step(m != 0).astype(jnp.int32)

    lax.while_loop(cond, body, step(active).astype(jnp.int32))


def _mst_body(w_hbm, out_hbm, w_v, comp_v, minidx_v, ptr_v, mark_v, sem):
    wid = lax.axis_index("s") * 2 + lax.axis_index("c")

    @pl.when(wid < _NTREE)
    def _run():
        pltpu.sync_copy(w_hbm.at[wid], w_v)
        iota = _lane_iota()
        sen_vec = jnp.full((16,), _SEN, jnp.int32)
        zero_vec = jnp.zeros((16,), jnp.int32)

        @plsc.parallel_loop(0, _N, 16, unroll=4)
        def _init_blk(b):
            ds = pl.ds(b, 16)
            comp_v[ds] = b + iota
            minidx_v[ds] = sen_vec
            mark_v[ds] = zero_vec
            mark_v[pl.ds(_N + b, 16)] = zero_vec

        def slack_blk(i, _):
            comp_v[pl.ds(_N + i * 16, 16)] = jnp.full((16,), _N - 1, jnp.int32)
            return 0

        lax.fori_loop(0, _W // 16, slack_blk, 0)

        def edge_h_blk(i, _):
            base = i * 16
            e = base + iota
            cu = comp_v[pl.ds(base, 16)]
            cv = comp_v[pl.ds(base + 1, 16)]
            we = w_v[pl.ds(base, 16)]
            active = cu != cv
            _scatter_min(minidx_v, w_v, cu, e, we, active)
            _scatter_min(minidx_v, w_v, cv, e, we, active)
            return 0

        def edge_v_blk(i, _):
            base = i * 16
            e = _N + base + iota
            cu = comp_v[pl.ds(base, 16)]
            cv = comp_v[pl.ds(base + _W, 16)]
            we = w_v[pl.ds(_N + base, 16)]
            active = cu != cv
            _scatter_min(minidx_v, w_v, cu, e, we, active)
            _scatter_min(minidx_v, w_v, cv, e, we, active)
            return 0

        def link_blk(b, acc):
            ds = pl.ds(b, 16)
            c = b + iota
            e = minidx_v[ds]
            has = e != _SEN
            plsc.store_scatter(mark_v, [e], jnp.ones((16,), jnp.int32),
                               mask=has)
            is_h = e < _N
            u = jnp.where(is_h, e, e - _N)
            v = jnp.where(is_h, e + 1, u + _W)
            v = jnp.minimum(v, _N - 1)
            cu = plsc.load_gather(comp_v, [u])
            cv = plsc.load_gather(comp_v, [v])
            other = jnp.where(cu == c, cv, cu)
            ptr_v[ds] = jnp.where(has, other, c)
            minidx_v[ds] = sen_vec
            return acc | has.astype(jnp.int32)

        def cycle_blk(b):
            ds = pl.ds(b, 16)
            c = b + iota
            p = ptr_v[ds]
            pp = plsc.load_gather(ptr_v, [p])
            ptr_v[ds] = jnp.where((pp == c) & (c < p), c, p)

        def jump_blk(b, acc):
            ds = pl.ds(b, 16)
            p = ptr_v[ds]
            pp = plsc.load_gather(ptr_v, [p])
            ptr_v[ds] = pp
            return acc | (pp != p).astype(jnp.int32)

        def jump_pass(_, changed):
            nj = jnp.where(changed > 0, _N, 0)
            acc = plsc.parallel_loop(0, nj, 16, unroll=4,
                                     carry=zero_vec)(jump_blk)
            return jnp.sum(acc)

        def comp_blk(b):
            ds = pl.ds(b, 16)
            comp_v[ds] = plsc.load_gather(ptr_v, [comp_v[ds]])

        def round_body(r, go):
            ne = jnp.where(go > 0, _NBLK_N, 0)
            nn = ne * 16
            lax.fori_loop(0, ne, edge_h_blk, 0)
            lax.fori_loop(0, ne, edge_v_blk, 0)
            merged = jnp.sum(
                plsc.parallel_loop(0, nn, 16, unroll=4,
                                   carry=zero_vec)(link_blk))
            plsc.parallel_loop(0, nn, 16, unroll=4)(cycle_blk)
            lax.fori_loop(0, jnp.where(merged > 0, 14 - r, 0), jump_pass, 1)
            plsc.parallel_loop(0, nn, 16, unroll=4)(comp_blk)
            return jnp.where(merged > 0, jnp.int32(1), jnp.int32(0))

        lax.fori_loop(0, 14, round_body, jnp.int32(1))
        pltpu.sync_copy(mark_v, out_hbm.at[wid])


def _mst_masks(w_all):
    mesh = plsc.VectorSubcoreMesh(core_axis_name="c", subcore_axis_name="s",
                                  num_cores=2, num_subcores=16)
    k = functools.partial(
        pl.kernel,
        out_type=jax.ShapeDtypeStruct((_NTREE, _M), jnp.int32),
        mesh=mesh,
        scratch_types=[
            pltpu.VMEM((_M,), jnp.float32),
            pltpu.VMEM((_N + _W,), jnp.int32),
            pltpu.VMEM((_N,), jnp.int32),
            pltpu.VMEM((_N,), jnp.int32),
            pltpu.VMEM((_M,), jnp.int32),
            pltpu.SemaphoreType.DMA,
        ],
        compiler_params=pltpu.CompilerParams(needs_layout_passes=False),
    )(_mst_body)
    return k(w_all)


def _agg3(x, ewh, ewv):
    t = ewh * x
    s = ewv * x
    return (ewh * jnp.roll(x, -1, axis=2) + jnp.roll(t, 1, axis=2)
            + ewv * jnp.roll(x, -1, axis=1) + jnp.roll(s, 1, axis=1))


def _agg2(x, ewh, ewv):
    t = ewh * x
    s = ewv * x
    return (ewh * jnp.roll(x, -1, axis=1) + jnp.roll(t, 1, axis=1)
            + ewv * jnp.roll(x, -1, axis=0) + jnp.roll(s, 1, axis=0))


def _tree_filter(feat, ewh, ewv):
    num = feat
    den = jnp.ones((_H, _W), jnp.float32)
    for _ in range(_ITERS):
        num = feat + _agg3(num, ewh, ewv)
        den = 1.0 + _agg2(den, ewh, ewv)
    return num / den[None]


def _loss_body(preds_ref, dl_ref, d1_ref, d2_ref, d3_ref,
               ml_ref, m1_ref, m2_ref, m3_ref, roi_ref,
               loss_ref, rsum_ref):
    logits = preds_ref[0]
    m = jnp.max(logits, axis=0, keepdims=True)
    ex = jnp.exp(logits - m)
    p = ex / jnp.sum(ex, axis=0, keepdims=True)

    def ew(d_ref, m_ref):
        d = d_ref[0]
        mk = m_ref[0]
        wh = jnp.exp(-d[0] / _SIGMA) * mk[0]
        wv = jnp.exp(-d[1] / _SIGMA) * mk[1]
        return wh, wv

    lh, lv = ew(dl_ref, ml_ref)
    a_s = _tree_filter(p, lh, lv)
    roi = roi_ref[0, 0]
    loss = jnp.float32(0.0)
    for dr, mr in ((d1_ref, m1_ref), (d2_ref, m2_ref), (d3_ref, m3_ref)):
        wh, wv = ew(dr, mr)
        ask = _tree_filter(a_s, wh, wv)
        loss = loss + jnp.sum(roi[None] * jnp.abs(p - ask))
    loss_ref[...] = jnp.full((1, 8, 128), loss)
    rsum_ref[...] = jnp.full((1, 8, 128), jnp.sum(roi))


def _loss_parts(preds, dists, masks, roi):
    b = preds.shape[0]
    spec4 = pl.BlockSpec((1, 2, _H, _W), lambda i: (i, 0, 0, 0))
    outs = pl.pallas_call(
        _loss_body,
        grid=(b,),
        in_specs=[pl.BlockSpec((1, preds.shape[1], _H, _W),
                               lambda i: (i, 0, 0, 0))]
        + [spec4] * 8
        + [pl.BlockSpec((1, 1, _H, _W), lambda i: (i, 0, 0, 0))],
        out_specs=[pl.BlockSpec((1, 8, 128), lambda i: (i, 0, 0))] * 2,
        out_shape=[jax.ShapeDtypeStruct((b, 8, 128), jnp.float32)] * 2,
    )(preds, *dists, *masks, roi)
    return outs


def _resize(x, h, w, method):
    return jax.image.resize(x, (x.shape[0], x.shape[1], h, w), method=method)


@jax.jit
def kernel(preds, low_feats, high_feats_1, high_feats_2, high_feats_3,
           unlabeled_ROIs, weight):
    b, c, h, w = preds.shape
    low_r = _resize(low_feats, h, w, 'bilinear')
    h1 = _resize(high_feats_1, h, w, 'bilinear')
    h2 = _resize(high_feats_2, h, w, 'bilinear')
    h3 = _resize(high_feats_3, h, w, 'bilinear')
    roi = jax.image.resize(unlabeled_ROIs[:, None].astype(jnp.float32),
                           (b, 1, h, w), method='nearest')

    d_low = _edge_dists(low_r)
    d_1 = _edge_dists(h1)
    d_2 = _edge_dists(h2)
    d_3 = _edge_dists(h3)

    w_all = jnp.concatenate([d_low, d_1, d_2, d_3], axis=0).reshape(_NTREE, _M)
    marks = _mst_masks(w_all).astype(jnp.float32).reshape(_NTREE, 2, _H, _W)
    m_low, m_1, m_2, m_3 = (marks[0:4], marks[4:8], marks[8:12], marks[12:16])

    loss_b, rsum_b = _loss_parts(
        preds, (d_low, d_1, d_2, d_3), (m_low, m_1, m_2, m_3), roi)
    loss = jnp.sum(loss_b[:, 0, 0])
    n = jnp.sum(rsum_b[:, 0, 0])
    loss = jnp.where(n > 0, loss / n, loss)
    return jnp.float32(weight) * loss
